# baseline (speedup 1.0000x reference)
import functools

import jax
import jax.numpy as jnp
from jax import lax
from jax.experimental import pallas as pl
from jax.experimental.pallas import tpu as pltpu
from jax.experimental.pallas import tpu_sc as plsc

S = 200
B = 4096
MAX_SEQ = 200
D = 64
SB = S * B
NC = 2
NS = 16
NW = NC * NS
C = 256
GSUB = 128
G = C // 16
ST = S // 8
BT = B // 128
NCHUNK = ST * BT * 4
EPS = 1e-5


def _splat(v, r):
    idx = jnp.full((16, 1), r, jnp.int32)
    dnums = lax.GatherDimensionNumbers(
        offset_dims=(), collapsed_slice_dims=(0,), start_index_map=(0,))
    return lax.gather(v, idx, dnums, (1,),
                      mode=lax.GatherScatterMode.PROMISE_IN_BOUNDS)


def _rsqrt(v):
    i = lax.bitcast_convert_type(v, jnp.int32)
    i = jnp.int32(0x5F3759DF) - (i >> 1)
    y = lax.bitcast_convert_type(i, jnp.float32)
    h = v * jnp.float32(0.5)
    for _ in range(3):
        y = y * (jnp.float32(1.5) - h * y * y)
    return y


def _body(seq_hbm, pid_hbm, item_hbm, pos_hbm, w_hbm, b_hbm, out_hbm,
          idx_a0, idx_a1, idx_p0, idx_p1, rows0, rows1, pos0, pos1,
          bufx0, bufx1, w_v, b_v, scr, scr2,
          sem_g0, sem_g1, sem_w0, sem_w1):
    wid = lax.axis_index("c") * NS + lax.axis_index("s")
    n0 = wid * (NCHUNK // NW)

    pltpu.sync_copy(w_hbm, w_v)
    pltpu.sync_copy(b_hbm, b_v)

    iota17 = lax.iota(jnp.int32, 16) * 17
    w_regs = [w_v[pl.ds(k * 16, 16)] for k in range(4)]
    b_regs = [b_v[pl.ds(k * 16, 16)] for k in range(4)]
    zero_f = jnp.zeros((16,), jnp.float32)

    idx_as = (idx_a0, idx_a1)
    idx_ps = (idx_p0, idx_p1)
    rows = (rows0, rows1)
    prows = (pos0, pos1)
    bufx = (bufx0, bufx1)
    sem_g = (sem_g0, sem_g1)
    sem_w = (sem_w0, sem_w1)

    def _coords(n):
        st = n // (BT * 4)
        rem = n % (BT * 4)
        return st, rem // 4, rem % 4

    def _load_idx(n, P):
        st, bt, q = _coords(n)
        pltpu.sync_copy(seq_hbm.at[st, bt, pl.ds(q * 2, 2)], idx_as[P])
        pltpu.sync_copy(pid_hbm.at[st, bt, pl.ds(q * 2, 2)], idx_ps[P])

    def _gather_cps(P):
        return [
            pltpu.make_async_copy(
                item_hbm.at[idx_as[P].at[j]],
                rows[P].at[pl.ds(j * GSUB, GSUB)],
                sem_g[P],
            )
            for j in range(2)
        ] + [
            pltpu.make_async_copy(
                pos_hbm.at[idx_ps[P].at[j]],
                prows[P].at[pl.ds(j * GSUB, GSUB)],
                sem_g[P],
            )
            for j in range(2)
        ]

    def _wb_cps(n, P):
        st, bt, q = _coords(n)
        cps = []
        for sj in range(2):
            srow = st * 8 + q * 2 + sj
            for dt in range(8):
                off = ((srow * 8 + dt) * BT + bt) * 1024
                cps.append(pltpu.make_async_copy(
                    bufx[P].at[pl.ds(sj * 8192 + dt * 1024, 1024)],
                    out_hbm.at[pl.ds(off, 1024)],
                    sem_w[P],
                ))
        return cps

    def _compute(rows_b, pos_b, x_b):
        def _group(g, carry):
            grow = g * 16

            for r in range(16):
                row = grow + r
                xs = []
                for k in range(4):
                    a = rows_b[row, pl.ds(k * 16, 16)]
                    pp = pos_b[row, pl.ds(k * 16, 16)]
                    xs.append(a * jnp.float32(8.0) + pp)
                pr = (xs[0] + xs[1]) + (xs[2] + xs[3])
                q0, q1, q2, q3 = (x * x for x in xs)
                qr = (q0 + q1) + (q2 + q3)
                scr[pl.ds(r * 17, 16)] = pr
                scr[pl.ds((16 + r) * 17, 16)] = qr

            s1a = s1b = s2a = s2b = zero_f
            for j in range(16):
                c1 = plsc.load_gather(scr, [iota17 + j])
                c2 = plsc.load_gather(scr, [iota17 + (16 * 17 + j)])
                if j % 2:
                    s1b = s1b + c1
                    s2b = s2b + c2
                else:
                    s1a = s1a + c1
                    s2a = s2a + c2
            s1 = s1a + s1b
            s2 = s2a + s2b
            m = s1 * jnp.float32(1.0 / D)
            var = s2 * jnp.float32(1.0 / D) - m * m + jnp.float32(EPS)
            sc = _rsqrt(var)
            u = m * sc

            for r in range(16):
                srv = _splat(sc, r)
                urv = _splat(u, r)
                row = grow + r
                for k in range(4):
                    a = rows_b[row, pl.ds(k * 16, 16)]
                    pp = pos_b[row, pl.ds(k * 16, 16)]
                    x = a * jnp.float32(8.0) + pp
                    scr2[pl.ds(k * 272 + r * 17, 16)] = (
                        (x * srv - urv) * w_regs[k] + b_regs[k])
            gb = (g // 8) * 8192 + (g % 8) * 16
            for k in range(4):
                for j in range(16):
                    d = k * 16 + j
                    colv = plsc.load_gather(scr2, [iota17 + (k * 272 + j)])
                    x_b[pl.ds(gb + (d // 8) * 1024 + (d % 8) * 128, 16)] = colv
            return carry

        lax.fori_loop(0, G, _group, 0)

    NCW = NCHUNK // NW
    _load_idx(n0, 0)
    for cp in _gather_cps(0):
        cp.start()

    def _step(ii, carry):
        for half in range(2):
            n = n0 + ii * 2 + half
            P = half
            for cp in _gather_cps(P):
                cp.wait()

            if half == 0:
                _load_idx(n + 1, 1 - P)
                for cp in _gather_cps(1 - P):
                    cp.start()
            else:
                @pl.when(ii < (NCW // 2) - 1)
                def _prefetch():
                    _load_idx(n + 1, 1 - P)
                    for cp in _gather_cps(1 - P):
                        cp.start()

            @pl.when(ii > 0)
            def _drain():
                for cp in _wb_cps(n, P):
                    cp.wait()

            _compute(rows[P], prows[P], bufx[P])
            for cp in _wb_cps(n, P):
                cp.start()
        return carry

    lax.fori_loop(0, NCW // 2, _step, 0)
    for cp in _wb_cps(n0, 0):
        cp.wait()
    for cp in _wb_cps(n0, 1):
        cp.wait()


@jax.jit
def _emb(seq_t, pid_t, item_table, pos_table, ln_weight, ln_bias):
    mesh = plsc.VectorSubcoreMesh(core_axis_name="c", subcore_axis_name="s")
    f = functools.partial(
        pl.kernel,
        out_type=jax.ShapeDtypeStruct((SB * D,), jnp.float32),
        mesh=mesh,
        scratch_types=[
            pltpu.VMEM((2, 128), jnp.int32),
            pltpu.VMEM((2, 128), jnp.int32),
            pltpu.VMEM((2, 128), jnp.int32),
            pltpu.VMEM((2, 128), jnp.int32),
            pltpu.VMEM((C, D), jnp.float32),
            pltpu.VMEM((C, D), jnp.float32),
            pltpu.VMEM((C, D), jnp.float32),
            pltpu.VMEM((C, D), jnp.float32),
            pltpu.VMEM((C * D,), jnp.float32),
            pltpu.VMEM((C * D,), jnp.float32),
            pltpu.VMEM((D,), jnp.float32),
            pltpu.VMEM((D,), jnp.float32),
            pltpu.VMEM((32 * 17,), jnp.float32),
            pltpu.VMEM((4 * 272,), jnp.float32),
            pltpu.SemaphoreType.DMA,
            pltpu.SemaphoreType.DMA,
            pltpu.SemaphoreType.DMA,
            pltpu.SemaphoreType.DMA,
        ],
        compiler_params=pltpu.CompilerParams(
            needs_layout_passes=False, use_tc_tiling_on_sc=False),
    )(_body)
    return f(seq_t, pid_t, item_table, pos_table, ln_weight, ln_bias)


def kernel(input_sequence, position_ids, item_table, pos_table, ln_weight, ln_bias):
    seq_t = input_sequence.reshape(ST, 8, BT, 128).transpose(0, 2, 1, 3)
    pid_t = position_ids.reshape(ST, 8, BT, 128).transpose(0, 2, 1, 3)
    out = _emb(seq_t, pid_t, item_table, pos_table, ln_weight, ln_bias)
    out5 = out.reshape(S, 8, BT, 8, 128)
    return out5.transpose(0, 2, 4, 1, 3).reshape(S, B, D)


# speedup vs baseline: 1.4505x; 1.4505x over previous
import functools

import jax
import jax.numpy as jnp
from jax import lax
from jax.experimental import pallas as pl
from jax.experimental.pallas import tpu as pltpu
from jax.experimental.pallas import tpu_sc as plsc

S = 200
B = 4096
MAX_SEQ = 200
D = 64
SB = S * B
NC = 2
NS = 16
NW = NC * NS
C = 256
GSUB = 128
G = C // 16
ST = S // 8
BT = B // 128
NCHUNK = ST * BT * 4
EPS = 1e-5


def _splat(v, r):
    idx = jnp.full((16, 1), r, jnp.int32)
    dnums = lax.GatherDimensionNumbers(
        offset_dims=(), collapsed_slice_dims=(0,), start_index_map=(0,))
    return lax.gather(v, idx, dnums, (1,),
                      mode=lax.GatherScatterMode.PROMISE_IN_BOUNDS)


def _rsqrt(v):
    i = lax.bitcast_convert_type(v, jnp.int32)
    i = jnp.int32(0x5F3759DF) - (i >> 1)
    y = lax.bitcast_convert_type(i, jnp.float32)
    h = v * jnp.float32(0.5)
    for _ in range(3):
        y = y * (jnp.float32(1.5) - h * y * y)
    return y


def _body(seq_hbm, pid_hbm, item_hbm, pos_hbm, w_hbm, b_hbm, out_hbm,
          idx_a0, idx_a1, idx_p0, idx_p1, rows0, rows1, pos0, pos1,
          bufx0, bufx1, w_v, b_v, scr,
          sem_g0, sem_g1, sem_w0, sem_w1):
    wid = lax.axis_index("c") * NS + lax.axis_index("s")
    n0 = wid * (NCHUNK // NW)

    pltpu.sync_copy(w_hbm, w_v)
    pltpu.sync_copy(b_hbm, b_v)

    iota17 = lax.iota(jnp.int32, 16) * 17
    w_regs = [w_v[pl.ds(k * 16, 16)] for k in range(4)]
    b_regs = [b_v[pl.ds(k * 16, 16)] for k in range(4)]
    zero_f = jnp.zeros((16,), jnp.float32)

    idx_as = (idx_a0, idx_a1)
    idx_ps = (idx_p0, idx_p1)
    rows = (rows0, rows1)
    prows = (pos0, pos1)
    bufx = (bufx0, bufx1)
    sem_g = (sem_g0, sem_g1)
    sem_w = (sem_w0, sem_w1)

    def _coords(n):
        st = n // (BT * 4)
        rem = n % (BT * 4)
        return st, rem // 4, rem % 4

    def _load_idx(n, P):
        st, bt, q = _coords(n)
        pltpu.sync_copy(seq_hbm.at[st, bt, pl.ds(q * 2, 2)], idx_as[P])
        pltpu.sync_copy(pid_hbm.at[st, bt, pl.ds(q * 2, 2)], idx_ps[P])

    def _gather_cps(P):
        return [
            pltpu.make_async_copy(
                item_hbm.at[idx_as[P].at[j]],
                rows[P].at[pl.ds(j * GSUB, GSUB)],
                sem_g[P],
            )
            for j in range(2)
        ] + [
            pltpu.make_async_copy(
                pos_hbm.at[idx_ps[P].at[j]],
                prows[P].at[pl.ds(j * GSUB, GSUB)],
                sem_g[P],
            )
            for j in range(2)
        ]

    def _wb_cps(n, P):
        st, bt, q = _coords(n)
        cps = []
        for sj in range(2):
            off = ((st * 8 + q * 2 + sj) * B + bt * 128) * D
            cps.append(pltpu.make_async_copy(
                bufx[P].at[pl.ds(sj * GSUB * D, GSUB * D)],
                out_hbm.at[pl.ds(off, GSUB * D)],
                sem_w[P],
            ))
        return cps

    def _compute(rows_b, pos_b, x_b):
        def _group(g, carry):
            grow = g * 16
            gx = grow * D

            for r in range(16):
                row = grow + r
                xs = []
                for k in range(4):
                    a = rows_b[row, pl.ds(k * 16, 16)]
                    pp = pos_b[row, pl.ds(k * 16, 16)]
                    xs.append(a * jnp.float32(8.0) + pp)
                pr = (xs[0] + xs[1]) + (xs[2] + xs[3])
                q0, q1, q2, q3 = (x * x for x in xs)
                qr = (q0 + q1) + (q2 + q3)
                scr[pl.ds(r * 17, 16)] = pr
                scr[pl.ds((16 + r) * 17, 16)] = qr

            s1a = s1b = s2a = s2b = zero_f
            for j in range(16):
                c1 = plsc.load_gather(scr, [iota17 + j])
                c2 = plsc.load_gather(scr, [iota17 + (16 * 17 + j)])
                if j % 2:
                    s1b = s1b + c1
                    s2b = s2b + c2
                else:
                    s1a = s1a + c1
                    s2a = s2a + c2
            s1 = s1a + s1b
            s2 = s2a + s2b
            m = s1 * jnp.float32(1.0 / D)
            var = s2 * jnp.float32(1.0 / D) - m * m + jnp.float32(EPS)
            sc = _rsqrt(var)
            u = m * sc

            for r in range(16):
                srv = _splat(sc, r)
                urv = _splat(u, r)
                row = grow + r
                xoff = gx + r * D
                for k in range(4):
                    a = rows_b[row, pl.ds(k * 16, 16)]
                    pp = pos_b[row, pl.ds(k * 16, 16)]
                    x = a * jnp.float32(8.0) + pp
                    x_b[pl.ds(xoff + k * 16, 16)] = (
                        (x * srv - urv) * w_regs[k] + b_regs[k])
            return carry

        lax.fori_loop(0, G, _group, 0)

    NCW = NCHUNK // NW
    _load_idx(n0, 0)
    for cp in _gather_cps(0):
        cp.start()

    def _step(ii, carry):
        for half in range(2):
            n = n0 + ii * 2 + half
            P = half
            for cp in _gather_cps(P):
                cp.wait()

            if half == 0:
                _load_idx(n + 1, 1 - P)
                for cp in _gather_cps(1 - P):
                    cp.start()
            else:
                @pl.when(ii < (NCW // 2) - 1)
                def _prefetch():
                    _load_idx(n + 1, 1 - P)
                    for cp in _gather_cps(1 - P):
                        cp.start()

            @pl.when(ii > 0)
            def _drain():
                for cp in _wb_cps(n, P):
                    cp.wait()

            _compute(rows[P], prows[P], bufx[P])
            for cp in _wb_cps(n, P):
                cp.start()
        return carry

    lax.fori_loop(0, NCW // 2, _step, 0)
    for cp in _wb_cps(n0, 0):
        cp.wait()
    for cp in _wb_cps(n0, 1):
        cp.wait()


@jax.jit
def _emb(seq_t, pid_t, item_table, pos_table, ln_weight, ln_bias):
    mesh = plsc.VectorSubcoreMesh(core_axis_name="c", subcore_axis_name="s")
    f = functools.partial(
        pl.kernel,
        out_type=jax.ShapeDtypeStruct((SB * D,), jnp.float32),
        mesh=mesh,
        scratch_types=[
            pltpu.VMEM((2, 128), jnp.int32),
            pltpu.VMEM((2, 128), jnp.int32),
            pltpu.VMEM((2, 128), jnp.int32),
            pltpu.VMEM((2, 128), jnp.int32),
            pltpu.VMEM((C, D), jnp.float32),
            pltpu.VMEM((C, D), jnp.float32),
            pltpu.VMEM((C, D), jnp.float32),
            pltpu.VMEM((C, D), jnp.float32),
            pltpu.VMEM((C * D,), jnp.float32),
            pltpu.VMEM((C * D,), jnp.float32),
            pltpu.VMEM((D,), jnp.float32),
            pltpu.VMEM((D,), jnp.float32),
            pltpu.VMEM((32 * 17,), jnp.float32),
            pltpu.SemaphoreType.DMA,
            pltpu.SemaphoreType.DMA,
            pltpu.SemaphoreType.DMA,
            pltpu.SemaphoreType.DMA,
        ],
        compiler_params=pltpu.CompilerParams(
            needs_layout_passes=False, use_tc_tiling_on_sc=False),
    )(_body)
    return f(seq_t, pid_t, item_table, pos_table, ln_weight, ln_bias)


def kernel(input_sequence, position_ids, item_table, pos_table, ln_weight, ln_bias):
    seq_t = input_sequence.reshape(ST, 8, BT, 128).transpose(0, 2, 1, 3)
    pid_t = position_ids.reshape(ST, 8, BT, 128).transpose(0, 2, 1, 3)
    out = _emb(seq_t, pid_t, item_table, pos_table, ln_weight, ln_bias)
    return out.reshape(S, B, D)
